# baseline (speedup 1.0000x reference)
import functools

import jax
import jax.numpy as jnp
from jax import lax
from jax.experimental import pallas as pl
from jax.experimental.pallas import tpu as pltpu
from jax.experimental.pallas import tpu_sc as plsc

B = 32
D = 64
T = 1024
K = 1024
_LANES = 16
_HALF = T // 2


def _tc_body(x_ref, emb_ref, idx_ref, lsum_ref):
    b = pl.program_id(0)
    xb = x_ref[0]
    emb = emb_ref[...]
    mm = lax.dot_general(emb, xb, (((1,), (0,)), ((), ())),
                         preferred_element_type=jnp.float32)
    en2 = jnp.sum(emb * emb, axis=1)
    xn2 = jnp.sum(xb * xb, axis=0)
    dist = (xn2[None, :] + en2[:, None]) - 2.0 * mm
    m = jnp.min(dist, axis=0)
    hit = dist == m[None, :]
    iota = lax.broadcasted_iota(jnp.int32, (K, T), 0)
    idxs = jnp.min(jnp.where(hit, iota, K), axis=0)
    idx_ref[0, 0, :] = idxs
    s = jnp.sum(m)
    prev = jnp.where(b == 0, 0.0, lsum_ref[0, 0])
    tot = prev + s
    lsum_ref[0, 0] = jnp.where(b == B - 1, tot * (2.0 / (B * T * D)), tot)


_tc_call = pl.pallas_call(
    _tc_body,
    grid=(B,),
    in_specs=[
        pl.BlockSpec((1, D, T), lambda i: (i, 0, 0)),
        pl.BlockSpec((K, D), lambda i: (0, 0)),
    ],
    out_specs=[
        pl.BlockSpec((1, 1, T), lambda i: (i, 0, 0)),
        pl.BlockSpec((1, 1), lambda i: (0, 0), memory_space=pltpu.SMEM),
    ],
    out_shape=[
        jax.ShapeDtypeStruct((B, 1, T), jnp.int32),
        jax.ShapeDtypeStruct((1, 1), jnp.float32),
    ],
)


@functools.cache
def _make_sc_gather():
    mesh = plsc.VectorSubcoreMesh(core_axis_name="c", subcore_axis_name="s")

    @functools.partial(
        pl.kernel,
        mesh=mesh,
        out_type=jax.ShapeDtypeStruct((B, D, T), jnp.float32),
        compiler_params=pltpu.CompilerParams(needs_layout_passes=False),
        scratch_types=[
            pltpu.VMEM((K * D,), jnp.float32),
            pltpu.VMEM((T,), jnp.int32),
            pltpu.VMEM((D, _HALF), jnp.float32),
        ],
    )
    def _sc_gather(idx_hbm, emb_hbm, out_hbm, emb_v, idx_v, out_v):
        c = lax.axis_index("c")
        s = lax.axis_index("s")
        b = s * 2 + c
        pltpu.sync_copy(emb_hbm, emb_v)
        pltpu.sync_copy(idx_hbm.at[b], idx_v)
        for h in range(2):
            def body(g, carry):
                tb = h * _HALF + g * _LANES
                idxv = idx_v[pl.ds(tb, _LANES)]
                base = idxv * D
                for d in range(D):
                    val = plsc.load_gather(emb_v, [base + d])
                    out_v[d, pl.ds(g * _LANES, _LANES)] = val
                return carry

            lax.fori_loop(0, _HALF // _LANES, body, 0)
            pltpu.sync_copy(out_v, out_hbm.at[b, :, pl.ds(h * _HALF, _HALF)])

    return _sc_gather


def kernel(x, embedding):
    idx3, lsum = _tc_call(x, embedding)
    indexes = jnp.reshape(idx3, (B, T))
    values = _make_sc_gather()(indexes, jnp.reshape(embedding, (K * D,)))
    loss = jnp.reshape(lsum, ())
    return (values, indexes, loss)


# speedup vs baseline: 1.4601x; 1.4601x over previous
import functools

import jax
import jax.numpy as jnp
from jax import lax
from jax.experimental import pallas as pl
from jax.experimental.pallas import tpu as pltpu
from jax.experimental.pallas import tpu_sc as plsc

B = 32
D = 64
T = 1024
K = 1024
_LANES = 16
_HALF = T // 2


def _tc_body(x_ref, emb_ref, idx_ref, lsum_ref):
    b = pl.program_id(0)
    xb = x_ref[0]
    emb = emb_ref[...]
    mm2 = lax.dot_general(emb + emb, xb, (((1,), (0,)), ((), ())),
                          preferred_element_type=jnp.float32)
    en2 = jnp.sum(emb * emb, axis=1)
    xn2 = jnp.sum(xb * xb, axis=0)
    dist = (xn2[None, :] + en2[:, None]) - mm2
    m = jnp.min(dist, axis=0)
    hit = dist == m[None, :]
    iota = lax.broadcasted_iota(jnp.int32, (K, T), 0)
    idxs = jnp.min(jnp.where(hit, iota, K), axis=0)
    idx_ref[0, 0, :] = idxs
    s = jnp.sum(m)
    prev = jnp.where(b == 0, 0.0, lsum_ref[0, 0])
    tot = prev + s
    lsum_ref[0, 0] = jnp.where(b == B - 1, tot * (2.0 / (B * T * D)), tot)


_tc_call = pl.pallas_call(
    _tc_body,
    grid=(B,),
    in_specs=[
        pl.BlockSpec((1, D, T), lambda i: (i, 0, 0)),
        pl.BlockSpec((K, D), lambda i: (0, 0)),
    ],
    out_specs=[
        pl.BlockSpec((1, 1, T), lambda i: (i, 0, 0)),
        pl.BlockSpec((1, 1), lambda i: (0, 0), memory_space=pltpu.SMEM),
    ],
    out_shape=[
        jax.ShapeDtypeStruct((B, 1, T), jnp.int32),
        jax.ShapeDtypeStruct((1, 1), jnp.float32),
    ],
)


@functools.cache
def _make_sc_gather():
    mesh = plsc.VectorSubcoreMesh(core_axis_name="c", subcore_axis_name="s")

    @functools.partial(
        pl.kernel,
        mesh=mesh,
        out_type=jax.ShapeDtypeStruct((B, D, T), jnp.float32),
        compiler_params=pltpu.CompilerParams(needs_layout_passes=False),
        scratch_types=[
            pltpu.VMEM((K * D,), jnp.float32),
            pltpu.VMEM((T,), jnp.int32),
            pltpu.VMEM((D, _HALF), jnp.float32),
        ],
    )
    def _sc_gather(idx_hbm, embt_hbm, out_hbm, emb_v, idx_v, out_v):
        c = lax.axis_index("c")
        s = lax.axis_index("s")
        b = s * 2 + c
        pltpu.sync_copy(embt_hbm, emb_v)
        pltpu.sync_copy(idx_hbm.at[b], idx_v)
        for h in range(2):
            @plsc.parallel_loop(0, _HALF // _LANES, unroll=2)
            def body(g):
                tb = h * _HALF + g * _LANES
                idxv = idx_v[pl.ds(tb, _LANES)]
                for d in range(D):
                    val = plsc.load_gather(emb_v, [idxv + d * K])
                    out_v[d, pl.ds(g * _LANES, _LANES)] = val

            pltpu.sync_copy(out_v, out_hbm.at[b, :, pl.ds(h * _HALF, _HALF)])

    return _sc_gather


def kernel(x, embedding):
    idx3, lsum = _tc_call(x, embedding)
    indexes = jnp.reshape(idx3, (B, T))
    embt = jnp.reshape(jnp.transpose(embedding), (D * K,))
    values = _make_sc_gather()(indexes, embt)
    loss = jnp.reshape(lsum, ())
    return (values, indexes, loss)
